# baseline (speedup 1.0000x reference)
import functools

import jax
import jax.numpy as jnp
from jax import lax
from jax.experimental import pallas as pl
from jax.experimental.pallas import tpu as pltpu
from jax.experimental.pallas import tpu_sc as plsc

N = 10000
E = 320000
DIM = 128
L = 4
NC = 10

NTILES = 16
SLICE = 640
NPAD = NTILES * SLICE
CHUNK = 128
NCH = E // CHUNK
CPTMAX = 160
NCHPAD = 2560
GRP = 20


def _sc_propagate(ei3, dst1d):
    mesh = plsc.VectorSubcoreMesh(core_axis_name="c", subcore_axis_name="s")

    @functools.partial(
        pl.kernel,
        out_type=jax.ShapeDtypeStruct((L, NPAD), jnp.float32),
        mesh=mesh,
        compiler_params=pltpu.CompilerParams(needs_layout_passes=False),
        scratch_types=[
            pltpu.VMEM((CPTMAX * CHUNK,), jnp.int32),
            pltpu.VMEM((CPTMAX, CHUNK), jnp.int32),
            pltpu.VMEM((CPTMAX, CHUNK), jnp.int32),
            pltpu.VMEM((CPTMAX * CHUNK,), jnp.float32),
            pltpu.VMEM((NPAD,), jnp.float32),
            pltpu.VMEM((SLICE,), jnp.float32),
            pltpu.VMEM((SLICE,), jnp.float32),
            pltpu.VMEM((SLICE,), jnp.float32),
            pltpu.VMEM((SLICE,), jnp.float32),
            pltpu.VMEM((CHUNK,), jnp.float32),
            pltpu.VMEM_SHARED((NPAD,), jnp.float32),
            pltpu.VMEM_SHARED((NPAD,), jnp.float32),
            pltpu.SemaphoreType.DMA,
        ],
    )
    def k(ei_h, dst1_h, vout_h, dst_i, dst_r, src_r, upd, w_full,
          n_sl, v_sl, a_sl, z_sl, drain_buf, acc_sh, w_sh, sem):
        c = lax.axis_index("c")
        s = lax.axis_index("s")

        @pl.when(c == 0)
        def _body():
            base = s * SLICE
            ones16 = jnp.full((16,), 1.0, jnp.float32)
            c0 = s * CPTMAX
            nj = jnp.where(s < 15, 160, 100)

            pltpu.sync_copy(ei_h.at[1, pl.ds(c0, CPTMAX)], dst_r)
            pltpu.sync_copy(ei_h.at[0, pl.ds(c0, CPTMAX)], src_r)
            pltpu.sync_copy(dst1_h.at[pl.ds(c0 * CHUNK, CPTMAX * CHUNK)],
                            dst_i)

            @pl.loop(0, SLICE // 16)
            def _(i):
                z_sl[pl.ds(i * 16, 16)] = jnp.zeros((16,), jnp.float32)

            def drain(count):
                @pl.loop(0, count)
                def _(j):
                    pltpu.make_async_copy(
                        vout_h.at[0, pl.ds(0, CHUNK)], drain_buf, sem).wait()

            pltpu.sync_copy(z_sl, acc_sh.at[pl.ds(base, SLICE)])

            @pl.loop(0, CPTMAX * CHUNK // 16)
            def _(i):
                upd[pl.ds(i * 16, 16)] = ones16

            plsc.subcore_barrier()

            ng = nj // GRP

            @pl.loop(0, ng)
            def _(g):
                for t in range(GRP):
                    j = g * GRP + t
                    pltpu.async_copy(upd.at[pl.ds(j * CHUNK, CHUNK)],
                                     acc_sh.at[dst_r.at[j]], sem, add=True)

                @pl.when(g > 0)
                def _():
                    drain(GRP)

            drain(GRP)
            plsc.subcore_barrier()

            pltpu.sync_copy(acc_sh.at[pl.ds(base, SLICE)], a_sl)

            @pl.loop(0, SLICE // 16)
            def _(i):
                d = jnp.maximum(a_sl[pl.ds(i * 16, 16)], 1.0)
                y = 1.0 / d
                hd = 0.5 * d
                for _it in range(22):
                    y = y * (1.5 - hd * y * y)
                n_sl[pl.ds(i * 16, 16)] = y
                v_sl[pl.ds(i * 16, 16)] = ones16

            pltpu.sync_copy(z_sl, acc_sh.at[pl.ds(base, SLICE)])
            pltpu.sync_copy(n_sl, w_sh.at[pl.ds(base, SLICE)])
            plsc.subcore_barrier()

            for m in range(L):
                pltpu.sync_copy(w_sh, w_full)

                @pl.loop(0, ng)
                def _(g):
                    for t in range(GRP):
                        j = g * GRP + t
                        for k8 in range(CHUNK // 16):
                            idxv = dst_i[pl.ds(j * CHUNK + k8 * 16, 16)]
                            vals = plsc.load_gather(w_full, [idxv])
                            upd[pl.ds(j * CHUNK + k8 * 16, 16)] = vals
                        pltpu.async_copy(upd.at[pl.ds(j * CHUNK, CHUNK)],
                                         acc_sh.at[src_r.at[j]], sem,
                                         add=True)

                    @pl.when(g > 0)
                    def _():
                        drain(GRP)

                drain(GRP)
                plsc.subcore_barrier()

                pltpu.sync_copy(acc_sh.at[pl.ds(base, SLICE)], a_sl)
                pltpu.sync_copy(z_sl, acc_sh.at[pl.ds(base, SLICE)])

                @pl.loop(0, SLICE // 16)
                def _(i):
                    vv = v_sl[pl.ds(i * 16, 16)]
                    nn = n_sl[pl.ds(i * 16, 16)]
                    aw = a_sl[pl.ds(i * 16, 16)]
                    ww = w_full[pl.ds(base + i * 16, 16)]
                    vv = vv + nn * (ww + aw)
                    gidx = base + i * 16 + lax.iota(jnp.int32, 16)
                    vv = jnp.where(gidx < N, vv, 0.0)
                    v_sl[pl.ds(i * 16, 16)] = vv
                    a_sl[pl.ds(i * 16, 16)] = nn * vv

                pltpu.sync_copy(v_sl, vout_h.at[m, pl.ds(base, SLICE)])
                if m < L - 1:
                    pltpu.sync_copy(a_sl, w_sh.at[pl.ds(base, SLICE)])
                    plsc.subcore_barrier()

    return k(ei3, dst1d)


def _tc_tail(vmat, feat_p, W_emb, b_emb, Wp, bp, Wr, br):
    BN = 512
    grid = NPAD // BN

    def kern(v_ref, f_ref, we_ref, be_ref, wp_ref, bp_ref, wr_ref, br_ref,
             out_ref, m_acc, s_acc):
        i = pl.program_id(0)

        @pl.when(i == 0)
        def _():
            m_acc[...] = jnp.zeros_like(m_acc)
            s_acc[...] = jnp.zeros_like(s_acc)

        vb = v_ref[...]
        fb = f_ref[...]
        rbk = lambda x: x.astype(jnp.bfloat16).astype(jnp.float32)
        vh = rbk(vb)
        vl = vb - vh
        part = (jnp.dot(vh, fb, preferred_element_type=jnp.float32)
                + jnp.dot(vl, fb, preferred_element_type=jnp.float32))
        colsum = jnp.sum(fb, axis=0, keepdims=True)
        m_acc[...] += jnp.concatenate(
            [colsum, part, jnp.zeros((3, DIM), jnp.float32)], axis=0)
        s_acc[...] += jnp.sum(vb, axis=1, keepdims=True)

        @pl.when(i == grid - 1)
        def _():
            svec = s_acc[...]
            s = [jnp.float32(N)] + [svec[m, 0] for m in range(L)]
            we = rbk(we_ref[...])
            be = be_ref[...]
            m8 = m_acc[...]

            def hidot(a, b):
                ah = rbk(a)
                return (jnp.dot(ah, b, preferred_element_type=jnp.float32)
                        + jnp.dot(a - ah, b,
                                  preferred_element_type=jnp.float32))

            Z = [hidot(m8[m:m + 1], we.T) + s[m] * be[None, :]
                 for m in range(L + 1)]
            score = jnp.dot(rbk(Z[0]), rbk(wr_ref[0]).T,
                            preferred_element_type=jnp.float32) \
                + br_ref[0][None, :]
            for li in range(1, L + 1):
                wp = rbk(wp_ref[li - 1])
                bpv = bp_ref[li - 1]
                Z = [hidot(Z[m + 1], wp.T) + s[m] * bpv[None, :]
                     for m in range(L + 1 - li)]
                score = score + jnp.dot(rbk(Z[0]), rbk(wr_ref[li]).T,
                                        preferred_element_type=jnp.float32) \
                    + br_ref[li][None, :]
            out_ref[...] = score

    cspec = lambda shape: pl.BlockSpec(shape, lambda i: tuple(0 for _ in shape))
    return pl.pallas_call(
        kern,
        grid=(grid,),
        in_specs=[
            pl.BlockSpec((L, BN), lambda i: (0, i)),
            pl.BlockSpec((BN, DIM), lambda i: (i, 0)),
            cspec((DIM, DIM)),
            cspec((DIM,)),
            cspec((L, DIM, DIM)),
            cspec((L, DIM)),
            cspec((L + 1, NC, DIM)),
            cspec((L + 1, NC)),
        ],
        out_specs=pl.BlockSpec((1, NC), lambda i: (0, 0)),
        out_shape=jax.ShapeDtypeStruct((1, NC), jnp.float32),
        scratch_shapes=[
            pltpu.VMEM((8, DIM), jnp.float32),
            pltpu.VMEM((L, 1), jnp.float32),
        ],
    )(vmat, feat_p, W_emb, b_emb, Wp, bp, Wr, br)


def kernel(feat, edge_index, e, snorm_n, snorm_e, W_emb, b_emb, Wp, bp, Wr,
           br):
    ei_p = jnp.concatenate(
        [edge_index, jnp.zeros((2, NCHPAD * CHUNK - E), jnp.int32)], axis=1)
    ei3 = ei_p.reshape(2, NCHPAD, CHUNK)
    dst1d = ei_p[1]
    rb = lambda x: x.astype(jnp.bfloat16).astype(jnp.float32)
    feat_p = jnp.pad(rb(feat), ((0, NPAD - N), (0, 0)))

    vmat = _sc_propagate(ei3, dst1d)
    return _tc_tail(vmat, feat_p, W_emb, b_emb, Wp, bp, Wr, br)


# speedup vs baseline: 1.0152x; 1.0152x over previous
import functools

import jax
import jax.numpy as jnp
from jax import lax
from jax.experimental import pallas as pl
from jax.experimental.pallas import tpu as pltpu
from jax.experimental.pallas import tpu_sc as plsc

N = 10000
E = 320000
DIM = 128
L = 4
NC = 10

NTILES = 16
SLICE = 640
NPAD = NTILES * SLICE
CHUNK = 128
NCH = E // CHUNK
CPTMAX = 160
NCHPAD = 2560
GRP = 10


def _sc_propagate(ei3, dst1d):
    mesh = plsc.VectorSubcoreMesh(core_axis_name="c", subcore_axis_name="s")

    @functools.partial(
        pl.kernel,
        out_type=jax.ShapeDtypeStruct((L, NPAD), jnp.float32),
        mesh=mesh,
        compiler_params=pltpu.CompilerParams(needs_layout_passes=False),
        scratch_types=[
            pltpu.VMEM((CPTMAX * CHUNK,), jnp.int32),
            pltpu.VMEM((CPTMAX, CHUNK), jnp.int32),
            pltpu.VMEM((CPTMAX, CHUNK), jnp.int32),
            pltpu.VMEM((CPTMAX * CHUNK,), jnp.float32),
            pltpu.VMEM((NPAD,), jnp.float32),
            pltpu.VMEM((SLICE,), jnp.float32),
            pltpu.VMEM((SLICE,), jnp.float32),
            pltpu.VMEM((SLICE,), jnp.float32),
            pltpu.VMEM((SLICE,), jnp.float32),
            pltpu.VMEM((CHUNK,), jnp.float32),
            pltpu.VMEM_SHARED((NPAD,), jnp.float32),
            pltpu.VMEM_SHARED((NPAD,), jnp.float32),
            pltpu.SemaphoreType.DMA,
        ],
    )
    def k(ei_h, dst1_h, vout_h, dst_i, dst_r, src_r, upd, w_full,
          n_sl, v_sl, a_sl, z_sl, drain_buf, acc_sh, w_sh, sem):
        c = lax.axis_index("c")
        s = lax.axis_index("s")

        @pl.when(c == 0)
        def _body():
            base = s * SLICE
            ones16 = jnp.full((16,), 1.0, jnp.float32)
            c0 = s * CPTMAX
            nj = jnp.where(s < 15, 160, 100)

            pltpu.sync_copy(ei_h.at[1, pl.ds(c0, CPTMAX)], dst_r)
            pltpu.sync_copy(ei_h.at[0, pl.ds(c0, CPTMAX)], src_r)
            pltpu.sync_copy(dst1_h.at[pl.ds(c0 * CHUNK, CPTMAX * CHUNK)],
                            dst_i)

            @pl.loop(0, SLICE // 16)
            def _(i):
                z_sl[pl.ds(i * 16, 16)] = jnp.zeros((16,), jnp.float32)

            def drain(count):
                @pl.loop(0, count)
                def _(j):
                    pltpu.make_async_copy(
                        vout_h.at[0, pl.ds(0, CHUNK)], drain_buf, sem).wait()

            pltpu.sync_copy(z_sl, acc_sh.at[pl.ds(base, SLICE)])

            @pl.loop(0, CPTMAX * CHUNK // 16)
            def _(i):
                upd[pl.ds(i * 16, 16)] = ones16

            plsc.subcore_barrier()

            ng = nj // GRP

            @pl.loop(0, ng)
            def _(g):
                for t in range(GRP):
                    j = g * GRP + t
                    pltpu.async_copy(upd.at[pl.ds(j * CHUNK, CHUNK)],
                                     acc_sh.at[dst_r.at[j]], sem, add=True)

                @pl.when(g > 0)
                def _():
                    drain(GRP)

            drain(GRP)
            plsc.subcore_barrier()

            pltpu.sync_copy(acc_sh.at[pl.ds(base, SLICE)], a_sl)

            @pl.loop(0, SLICE // 16)
            def _(i):
                d = jnp.maximum(a_sl[pl.ds(i * 16, 16)], 1.0)
                y = 1.0 / d
                hd = 0.5 * d
                for _it in range(22):
                    y = y * (1.5 - hd * y * y)
                n_sl[pl.ds(i * 16, 16)] = y
                v_sl[pl.ds(i * 16, 16)] = ones16

            pltpu.sync_copy(z_sl, acc_sh.at[pl.ds(base, SLICE)])
            pltpu.sync_copy(n_sl, w_sh.at[pl.ds(base, SLICE)])
            plsc.subcore_barrier()

            for m in range(L):
                pltpu.sync_copy(w_sh, w_full)

                @pl.loop(0, ng)
                def _(g):
                    for t in range(GRP):
                        j = g * GRP + t
                        for k8 in range(CHUNK // 16):
                            idxv = dst_i[pl.ds(j * CHUNK + k8 * 16, 16)]
                            vals = plsc.load_gather(w_full, [idxv])
                            upd[pl.ds(j * CHUNK + k8 * 16, 16)] = vals
                        pltpu.async_copy(upd.at[pl.ds(j * CHUNK, CHUNK)],
                                         acc_sh.at[src_r.at[j]], sem,
                                         add=True)

                    @pl.when(g > 0)
                    def _():
                        drain(GRP)

                drain(GRP)
                plsc.subcore_barrier()

                pltpu.sync_copy(acc_sh.at[pl.ds(base, SLICE)], a_sl)
                pltpu.sync_copy(z_sl, acc_sh.at[pl.ds(base, SLICE)])

                @pl.loop(0, SLICE // 16)
                def _(i):
                    vv = v_sl[pl.ds(i * 16, 16)]
                    nn = n_sl[pl.ds(i * 16, 16)]
                    aw = a_sl[pl.ds(i * 16, 16)]
                    ww = w_full[pl.ds(base + i * 16, 16)]
                    vv = vv + nn * (ww + aw)
                    gidx = base + i * 16 + lax.iota(jnp.int32, 16)
                    vv = jnp.where(gidx < N, vv, 0.0)
                    v_sl[pl.ds(i * 16, 16)] = vv
                    a_sl[pl.ds(i * 16, 16)] = nn * vv

                pltpu.sync_copy(v_sl, vout_h.at[m, pl.ds(base, SLICE)])
                if m < L - 1:
                    pltpu.sync_copy(a_sl, w_sh.at[pl.ds(base, SLICE)])
                    plsc.subcore_barrier()

    return k(ei3, dst1d)


def _tc_tail(vmat, feat_p, W_emb, b_emb, Wp, bp, Wr, br):
    BN = 512
    grid = NPAD // BN

    def kern(v_ref, f_ref, we_ref, be_ref, wp_ref, bp_ref, wr_ref, br_ref,
             out_ref, m_acc, s_acc):
        i = pl.program_id(0)

        @pl.when(i == 0)
        def _():
            m_acc[...] = jnp.zeros_like(m_acc)
            s_acc[...] = jnp.zeros_like(s_acc)

        vb = v_ref[...]
        fb = f_ref[...]
        rbk = lambda x: x.astype(jnp.bfloat16).astype(jnp.float32)
        vh = rbk(vb)
        vl = vb - vh
        part = (jnp.dot(vh, fb, preferred_element_type=jnp.float32)
                + jnp.dot(vl, fb, preferred_element_type=jnp.float32))
        colsum = jnp.sum(fb, axis=0, keepdims=True)
        m_acc[...] += jnp.concatenate(
            [colsum, part, jnp.zeros((3, DIM), jnp.float32)], axis=0)
        s_acc[...] += jnp.sum(vb, axis=1, keepdims=True)

        @pl.when(i == grid - 1)
        def _():
            svec = s_acc[...]
            s = [jnp.float32(N)] + [svec[m, 0] for m in range(L)]
            we = rbk(we_ref[...])
            be = be_ref[...]
            m8 = m_acc[...]

            def hidot(a, b):
                ah = rbk(a)
                return (jnp.dot(ah, b, preferred_element_type=jnp.float32)
                        + jnp.dot(a - ah, b,
                                  preferred_element_type=jnp.float32))

            Z = [hidot(m8[m:m + 1], we.T) + s[m] * be[None, :]
                 for m in range(L + 1)]
            score = jnp.dot(rbk(Z[0]), rbk(wr_ref[0]).T,
                            preferred_element_type=jnp.float32) \
                + br_ref[0][None, :]
            for li in range(1, L + 1):
                wp = rbk(wp_ref[li - 1])
                bpv = bp_ref[li - 1]
                Z = [hidot(Z[m + 1], wp.T) + s[m] * bpv[None, :]
                     for m in range(L + 1 - li)]
                score = score + jnp.dot(rbk(Z[0]), rbk(wr_ref[li]).T,
                                        preferred_element_type=jnp.float32) \
                    + br_ref[li][None, :]
            out_ref[...] = score

    cspec = lambda shape: pl.BlockSpec(shape, lambda i: tuple(0 for _ in shape))
    return pl.pallas_call(
        kern,
        grid=(grid,),
        in_specs=[
            pl.BlockSpec((L, BN), lambda i: (0, i)),
            pl.BlockSpec((BN, DIM), lambda i: (i, 0)),
            cspec((DIM, DIM)),
            cspec((DIM,)),
            cspec((L, DIM, DIM)),
            cspec((L, DIM)),
            cspec((L + 1, NC, DIM)),
            cspec((L + 1, NC)),
        ],
        out_specs=pl.BlockSpec((1, NC), lambda i: (0, 0)),
        out_shape=jax.ShapeDtypeStruct((1, NC), jnp.float32),
        scratch_shapes=[
            pltpu.VMEM((8, DIM), jnp.float32),
            pltpu.VMEM((L, 1), jnp.float32),
        ],
    )(vmat, feat_p, W_emb, b_emb, Wp, bp, Wr, br)


def kernel(feat, edge_index, e, snorm_n, snorm_e, W_emb, b_emb, Wp, bp, Wr,
           br):
    ei_p = jnp.concatenate(
        [edge_index, jnp.zeros((2, NCHPAD * CHUNK - E), jnp.int32)], axis=1)
    ei3 = ei_p.reshape(2, NCHPAD, CHUNK)
    dst1d = ei_p[1]
    rb = lambda x: x.astype(jnp.bfloat16).astype(jnp.float32)
    feat_p = jnp.pad(rb(feat), ((0, NPAD - N), (0, 0)))

    vmat = _sc_propagate(ei3, dst1d)
    return _tc_tail(vmat, feat_p, W_emb, b_emb, Wp, bp, Wr, br)
